# baseline (speedup 1.0000x reference)
import jax
import jax.numpy as jnp
from jax import lax
from jax.experimental import pallas as pl
from jax.experimental.pallas import tpu as pltpu

_R = 4096
_C = 4096
_BR = 64
_NBUF = 4
_NCHUNKS = _R // _BR
_ROUNDS = _NCHUNKS // _NBUF


def _body(t_ref, w_hbm, o_hbm, ibuf, obuf, in_sem, out_sem):
    s = jax.nn.sigmoid(t_ref[0, 0])

    def in_copy(g, b):
        return pltpu.make_async_copy(
            w_hbm.at[pl.ds(g * _BR, _BR), :], ibuf.at[b], in_sem.at[b])

    def out_copy(g, b):
        return pltpu.make_async_copy(
            obuf.at[b], o_hbm.at[pl.ds(g * _BR, _BR), :], out_sem.at[b])

    def compute(b):
        w = ibuf[b]
        obuf[b] = w - jnp.minimum(jnp.maximum(w, -s), s)

    for b in range(_NBUF):
        in_copy(b, b).start()

    for b in range(_NBUF):
        in_copy(b, b).wait()
        compute(b)
        out_copy(b, b).start()
        in_copy(b + _NBUF, b).start()

    def round_body(r, _):
        for b in range(_NBUF):
            g = r * _NBUF + b
            in_copy(g, b).wait()
            out_copy(g - _NBUF, b).wait()
            compute(b)
            out_copy(g, b).start()

            @pl.when(g + _NBUF < _NCHUNKS)
            def _():
                in_copy(g + _NBUF, b).start()

        return _

    lax.fori_loop(1, _ROUNDS, round_body, None)

    for b in range(_NBUF):
        out_copy(_NCHUNKS - _NBUF + b, b).wait()


def kernel(weight, threshold, alpha):
    return pl.pallas_call(
        _body,
        in_specs=[
            pl.BlockSpec(memory_space=pltpu.SMEM),
            pl.BlockSpec(memory_space=pl.ANY),
        ],
        out_specs=pl.BlockSpec(memory_space=pl.ANY),
        out_shape=jax.ShapeDtypeStruct((_R, _C), jnp.float32),
        scratch_shapes=[
            pltpu.VMEM((_NBUF, _BR, _C), jnp.float32),
            pltpu.VMEM((_NBUF, _BR, _C), jnp.float32),
            pltpu.SemaphoreType.DMA((_NBUF,)),
            pltpu.SemaphoreType.DMA((_NBUF,)),
        ],
        compiler_params=pltpu.CompilerParams(
            vmem_limit_bytes=128 * 1024 * 1024,
        ),
    )(threshold, weight)


# speedup vs baseline: 1.0402x; 1.0402x over previous
import jax
import jax.numpy as jnp
from jax import lax
from jax.experimental import pallas as pl
from jax.experimental.pallas import tpu as pltpu

_R = 4096
_C = 4096
_BR = 256
_NBUF = 4
_NCHUNKS = _R // _BR
_ROUNDS = _NCHUNKS // _NBUF


def _body(t_ref, w_hbm, o_hbm, ibuf, obuf, in_sem, out_sem):
    s = jax.nn.sigmoid(t_ref[0, 0])

    def in_copy(g, b):
        return pltpu.make_async_copy(
            w_hbm.at[pl.ds(g * _BR, _BR), :], ibuf.at[b], in_sem.at[b])

    def out_copy(g, b):
        return pltpu.make_async_copy(
            obuf.at[b], o_hbm.at[pl.ds(g * _BR, _BR), :], out_sem.at[b])

    def compute(b):
        w = ibuf[b]
        obuf[b] = w - jnp.minimum(jnp.maximum(w, -s), s)

    for b in range(_NBUF):
        in_copy(b, b).start()

    for b in range(_NBUF):
        in_copy(b, b).wait()
        compute(b)
        out_copy(b, b).start()
        in_copy(b + _NBUF, b).start()

    def round_body(r, _):
        for b in range(_NBUF):
            g = r * _NBUF + b
            in_copy(g, b).wait()
            out_copy(g - _NBUF, b).wait()
            compute(b)
            out_copy(g, b).start()

            @pl.when(g + _NBUF < _NCHUNKS)
            def _():
                in_copy(g + _NBUF, b).start()

        return _

    lax.fori_loop(1, _ROUNDS, round_body, None)

    for b in range(_NBUF):
        out_copy(_NCHUNKS - _NBUF + b, b).wait()


def kernel(weight, threshold, alpha):
    return pl.pallas_call(
        _body,
        in_specs=[
            pl.BlockSpec(memory_space=pltpu.SMEM),
            pl.BlockSpec(memory_space=pl.ANY),
        ],
        out_specs=pl.BlockSpec(memory_space=pl.ANY),
        out_shape=jax.ShapeDtypeStruct((_R, _C), jnp.float32),
        scratch_shapes=[
            pltpu.VMEM((_NBUF, _BR, _C), jnp.float32),
            pltpu.VMEM((_NBUF, _BR, _C), jnp.float32),
            pltpu.SemaphoreType.DMA((_NBUF,)),
            pltpu.SemaphoreType.DMA((_NBUF,)),
        ],
        compiler_params=pltpu.CompilerParams(
            vmem_limit_bytes=128 * 1024 * 1024,
        ),
    )(threshold, weight)
